# baseline (speedup 1.0000x reference)
import jax
import jax.numpy as jnp
import numpy as np
from jax import lax
from jax.experimental import pallas as pl
from jax.experimental.pallas import tpu as pltpu
from jax.experimental.pallas import tpu_sc as plsc

EMB = 64
HIST = 50
ROW = 1 + HIST
PAIR = 2 * ROW
PADW = 128
PADP = 112
NC, NS = 2, 16
NW = NC * NS
LANES = 16
NCH = EMB // LANES
UCH = 128

_DCOL = np.arange(PADP).reshape(PADP // LANES, LANES)
_HALF = (_DCOL >= ROW).astype(np.int32)
_CSRC = (_HALF * PADW + (_DCOL - _HALF * ROW)).astype(np.int32)


def _sc_gather_pool(ninp, user_emb, rest_emb):
    B = ninp.shape[0]
    BPW = B // NW
    NP = BPW // 2
    NQ = BPW // UCH
    PPQ = UCH // 2

    def body(nin_hbm, user_hbm, rest_emb_hbm, out_u, out_s,
             slab_v, nin_v, uidx_v, acc, ubuf, bufA, bufB, semA, semB, semU):
        rest_hbm = rest_emb_hbm
        wid = lax.axis_index("s") * NC + lax.axis_index("c")
        base = wid * BPW
        lane = lax.iota(jnp.int32, LANES)
        zero16 = jnp.zeros((LANES,), jnp.int32)
        chalf, ccol = [], []
        for c in range(PADP // LANES):
            dcol = lane + c * LANES
            half = jnp.where(dcol >= ROW, 1, 0).astype(jnp.int32)
            chalf.append(half)
            ccol.append(dcol - half * ROW)

        for q in range(NQ):
            pltpu.sync_copy(nin_hbm.at[pl.ds(base + q * UCH, UCH)], slab_v)

            @pl.loop(0, PPQ, unroll=2)
            def _(p):
                dbase = (q * PPQ + p) * PADP
                for c in range(PADP // LANES):
                    v = plsc.load_gather(slab_v, [2 * p + chalf[c], ccol[c]])
                    nin_v[pl.ds(dbase + c * LANES, LANES)] = v

            for m in range(UCH // LANES):
                u16 = plsc.load_gather(slab_v, [m * LANES + lane, zero16])
                uidx_v[q, pl.ds(m * LANES, LANES)] = u16

        for q in range(NQ):
            pltpu.async_copy(user_hbm.at[uidx_v.at[q]],
                             ubuf.at[pl.ds(q * UCH, UCH)], semU)

        bufs = (bufA, bufB)
        sems = (semA, semB)
        pltpu.async_copy(rest_hbm.at[nin_v.at[pl.ds(0, PAIR)]], bufA, semA)

        @pl.loop(0, NP, step=2)
        def _(g):
            for b in range(2):
                t = g + b
                nb = (b + 1) % 2

                @pl.when(t + 1 < NP)
                def _():
                    pltpu.async_copy(
                        rest_hbm.at[nin_v.at[pl.ds((t + 1) * PADP, PAIR)]],
                        bufs[nb], sems[nb])

                pltpu.make_async_copy(
                    rest_hbm.at[nin_v.at[pl.ds(t * PADP, PAIR)]],
                    bufs[b], sems[b]).wait()
                buf = bufs[b]
                for half in range(2):
                    rb = half * ROW
                    arow = 2 * t + half
                    for c in range(NCH):
                        sl = pl.ds(c * LANES, LANES)
                        acc[arow, sl] = buf[rb + 1, sl]

                    @pl.loop(2, ROW, unroll=7)
                    def _(r):
                        for c in range(NCH):
                            sl = pl.ds(c * LANES, LANES)
                            plsc.addupdate(acc.at[arow, sl], buf[rb + r, sl])

        s_out = pltpu.async_copy(acc, out_s.at[pl.ds(base, BPW)], semA)
        for q in range(NQ):
            pltpu.make_async_copy(user_hbm.at[uidx_v.at[q]],
                                  ubuf.at[pl.ds(q * UCH, UCH)], semU).wait()
        pltpu.sync_copy(ubuf, out_u.at[pl.ds(base, BPW)])
        s_out.wait()

    f = pl.kernel(
        body,
        out_type=(jax.ShapeDtypeStruct((B, EMB), jnp.float32),
                  jax.ShapeDtypeStruct((B, EMB), jnp.float32)),
        mesh=plsc.VectorSubcoreMesh(core_axis_name="c", subcore_axis_name="s"),
        compiler_params=pltpu.CompilerParams(
            use_tc_tiling_on_sc=False, needs_layout_passes=False),
        scratch_types=[
            pltpu.VMEM((UCH, PADW), jnp.int32),
            pltpu.VMEM((B // NW // 2 * PADP,), jnp.int32),
            pltpu.VMEM((B // NW // UCH, UCH), jnp.int32),
            pltpu.VMEM((B // NW, EMB), jnp.float32),
            pltpu.VMEM((B // NW, EMB), jnp.float32),
            pltpu.VMEM((PAIR, EMB), jnp.float32),
            pltpu.VMEM((PAIR, EMB), jnp.float32),
            pltpu.SemaphoreType.DMA,
            pltpu.SemaphoreType.DMA,
            pltpu.SemaphoreType.DMA,
        ],
    )
    return f(ninp, user_emb, rest_emb)


def _mlp_body(u_ref, s_ref, w1u_ref, w1r_ref, b1_ref, w2_ref, b2_ref,
              w3_ref, b3_ref, o_ref):
    h1 = jnp.dot(u_ref[...], w1u_ref[...], preferred_element_type=jnp.float32)
    h1 += jnp.dot(s_ref[...], w1r_ref[...],
                  preferred_element_type=jnp.float32) * (1.0 / HIST)
    h1 = jnp.maximum(h1 + b1_ref[...], 0.0)
    h2 = jnp.dot(h1, w2_ref[...], preferred_element_type=jnp.float32)
    h2 = jnp.maximum(h2 + b2_ref[...], 0.0)
    y = jnp.dot(h2, w3_ref[...], preferred_element_type=jnp.float32)
    o_ref[...] = jax.nn.sigmoid(y + b3_ref[...])


def _tc_mlp(u, s, W1, b1, W2, b2, W3, b3):
    B = u.shape[0]
    H1, H2 = W1.shape[0], W2.shape[0]
    BLK = 2048
    grid = (B // BLK,)
    w1u = W1[:, :EMB].T
    w1r = W1[:, EMB:].T
    fixed = lambda i: (0, 0)
    return pl.pallas_call(
        _mlp_body,
        grid=grid,
        in_specs=[
            pl.BlockSpec((BLK, EMB), lambda i: (i, 0)),
            pl.BlockSpec((BLK, EMB), lambda i: (i, 0)),
            pl.BlockSpec((EMB, H1), fixed),
            pl.BlockSpec((EMB, H1), fixed),
            pl.BlockSpec((1, H1), fixed),
            pl.BlockSpec((H1, H2), fixed),
            pl.BlockSpec((1, H2), fixed),
            pl.BlockSpec((H2, 1), fixed),
            pl.BlockSpec((1, 1), fixed),
        ],
        out_specs=pl.BlockSpec((BLK, 1), lambda i: (i, 0)),
        out_shape=jax.ShapeDtypeStruct((B, 1), jnp.float32),
        compiler_params=pltpu.CompilerParams(
            dimension_semantics=("parallel",)),
    )(u, s, w1u, w1r, b1[None, :], W2.T, b2[None, :], W3.T, b3[None, :])


def kernel(net_input, user_emb, rest_emb, W1, b1, W2, b2, W3, b3):
    B = net_input.shape[0]
    ninp = jnp.pad(net_input, ((0, 0), (0, PADW - ROW)))
    u, s = _sc_gather_pool(ninp, user_emb, rest_emb)
    return _tc_mlp(u, s, W1, b1, W2, b2, W3, b3)


# speedup vs baseline: 1.0449x; 1.0449x over previous
import jax
import jax.numpy as jnp
import numpy as np
from jax import lax
from jax.experimental import pallas as pl
from jax.experimental.pallas import tpu as pltpu
from jax.experimental.pallas import tpu_sc as plsc

EMB = 64
HIST = 50
ROW = 1 + HIST
PAIR = 2 * ROW
PADW = 128
PADP = 112
NC, NS = 2, 16
NW = NC * NS
LANES = 16
NCH = EMB // LANES
UCH = 128

_DCOL = np.arange(PADP).reshape(PADP // LANES, LANES)
_HALF = (_DCOL >= ROW).astype(np.int32)
_CSRC = (_HALF * PADW + (_DCOL - _HALF * ROW)).astype(np.int32)


def _sc_gather_pool(ninp, user_p, rest_p):
    B = ninp.shape[0]
    BPW = B // NW
    NP = BPW // 2
    NQ = BPW // UCH
    PPQ = UCH // 2

    def body(nin_hbm, user_hbm, rest_hbm, out_u, out_s,
             slab_v, nin_v, uidx_v, acc, bufA, bufB, semA, semB, semU):
        wid = lax.axis_index("s") * NC + lax.axis_index("c")
        base = wid * BPW
        lane = lax.iota(jnp.int32, LANES)
        zero16 = jnp.zeros((LANES,), jnp.int32)
        chalf, ccol = [], []
        for c in range(PADP // LANES):
            dcol = lane + c * LANES
            half = jnp.where(dcol >= ROW, 1, 0).astype(jnp.int32)
            chalf.append(half)
            ccol.append(dcol - half * ROW)

        for q in range(NQ):
            pltpu.sync_copy(nin_hbm.at[pl.ds(base + q * UCH, UCH)], slab_v)

            @pl.loop(0, PPQ, unroll=2)
            def _(p):
                dbase = (q * PPQ + p) * PADP
                for c in range(PADP // LANES):
                    v = plsc.load_gather(slab_v, [2 * p + chalf[c], ccol[c]])
                    nin_v[pl.ds(dbase + c * LANES, LANES)] = v

            for m in range(UCH // LANES):
                u16 = plsc.load_gather(slab_v, [m * LANES + lane, zero16])
                uidx_v[q, pl.ds(m * LANES, LANES)] = u16

        bufs = (bufA, bufB)
        sems = (semA, semB)
        pltpu.async_copy(user_hbm.at[uidx_v.at[0]], bufA, semA)
        for q in range(NQ):
            b = q % 2
            if q + 1 < NQ:
                pltpu.async_copy(user_hbm.at[uidx_v.at[q + 1]],
                                 bufs[(q + 1) % 2], sems[(q + 1) % 2])
            pltpu.make_async_copy(user_hbm.at[uidx_v.at[q]],
                                  bufs[b], sems[b]).wait()
            pltpu.sync_copy(bufs[b].at[pl.ds(0, UCH), pl.ds(0, EMB)],
                            out_u.at[pl.ds(base + q * UCH, UCH)])

        pltpu.async_copy(rest_hbm.at[nin_v.at[pl.ds(0, PAIR)]],
                         bufA.at[pl.ds(0, PAIR)], semA)

        @pl.loop(0, NP, step=2)
        def _(g):
            for b in range(2):
                t = g + b
                nb = (b + 1) % 2

                @pl.when(t + 1 < NP)
                def _():
                    pltpu.async_copy(
                        rest_hbm.at[nin_v.at[pl.ds((t + 1) * PADP, PAIR)]],
                        bufs[nb].at[pl.ds(0, PAIR)], sems[nb])

                pltpu.make_async_copy(
                    rest_hbm.at[nin_v.at[pl.ds(t * PADP, PAIR)]],
                    bufs[b].at[pl.ds(0, PAIR)], sems[b]).wait()
                buf = bufs[b]
                for half in range(2):
                    rb = half * ROW
                    arow = 2 * t + half
                    for c in range(NCH):
                        sl = pl.ds(c * LANES, LANES)
                        acc[arow, sl] = buf[rb + 1, sl]

                    @pl.loop(2, ROW, unroll=7)
                    def _(r):
                        for c in range(NCH):
                            sl = pl.ds(c * LANES, LANES)
                            plsc.addupdate(acc.at[arow, sl], buf[rb + r, sl])

        pltpu.async_copy(acc, out_s.at[pl.ds(base, BPW)], semU).wait()

    f = pl.kernel(
        body,
        out_type=(jax.ShapeDtypeStruct((B, EMB), jnp.float32),
                  jax.ShapeDtypeStruct((B, EMB), jnp.float32)),
        mesh=plsc.VectorSubcoreMesh(core_axis_name="c", subcore_axis_name="s"),
        compiler_params=pltpu.CompilerParams(
            use_tc_tiling_on_sc=False, needs_layout_passes=False),
        scratch_types=[
            pltpu.VMEM((UCH, PADW), jnp.int32),
            pltpu.VMEM((B // NW // 2 * PADP,), jnp.int32),
            pltpu.VMEM((B // NW // UCH, UCH), jnp.int32),
            pltpu.VMEM((B // NW, EMB), jnp.float32),
            pltpu.VMEM((PAIR + 26, PADW), jnp.float32),
            pltpu.VMEM((PAIR + 26, PADW), jnp.float32),
            pltpu.SemaphoreType.DMA,
            pltpu.SemaphoreType.DMA,
            pltpu.SemaphoreType.DMA,
        ],
    )
    return f(ninp, user_p, rest_p)


def _mlp_body(u_ref, s_ref, w1u_ref, w1r_ref, b1_ref, w2_ref, b2_ref,
              w3_ref, b3_ref, o_ref):
    h1 = jnp.dot(u_ref[...], w1u_ref[...], preferred_element_type=jnp.float32)
    h1 += jnp.dot(s_ref[...], w1r_ref[...],
                  preferred_element_type=jnp.float32) * (1.0 / HIST)
    h1 = jnp.maximum(h1 + b1_ref[...], 0.0)
    h2 = jnp.dot(h1, w2_ref[...], preferred_element_type=jnp.float32)
    h2 = jnp.maximum(h2 + b2_ref[...], 0.0)
    y = jnp.dot(h2, w3_ref[...], preferred_element_type=jnp.float32)
    o_ref[...] = jax.nn.sigmoid(y + b3_ref[...])


def _tc_mlp(u, s, W1, b1, W2, b2, W3, b3):
    B = u.shape[0]
    H1, H2 = W1.shape[0], W2.shape[0]
    BLK = 2048
    grid = (B // BLK,)
    w1u = W1[:, :EMB].T
    w1r = W1[:, EMB:].T
    fixed = lambda i: (0, 0)
    return pl.pallas_call(
        _mlp_body,
        grid=grid,
        in_specs=[
            pl.BlockSpec((BLK, EMB), lambda i: (i, 0)),
            pl.BlockSpec((BLK, EMB), lambda i: (i, 0)),
            pl.BlockSpec((EMB, H1), fixed),
            pl.BlockSpec((EMB, H1), fixed),
            pl.BlockSpec((1, H1), fixed),
            pl.BlockSpec((H1, H2), fixed),
            pl.BlockSpec((1, H2), fixed),
            pl.BlockSpec((H2, 1), fixed),
            pl.BlockSpec((1, 1), fixed),
        ],
        out_specs=pl.BlockSpec((BLK, 1), lambda i: (i, 0)),
        out_shape=jax.ShapeDtypeStruct((B, 1), jnp.float32),
        compiler_params=pltpu.CompilerParams(
            dimension_semantics=("parallel",)),
    )(u, s, w1u, w1r, b1[None, :], W2.T, b2[None, :], W3.T, b3[None, :])


def kernel(net_input, user_emb, rest_emb, W1, b1, W2, b2, W3, b3):
    B = net_input.shape[0]
    ninp = jnp.pad(net_input, ((0, 0), (0, PADW - ROW)))
    user_p = jnp.pad(user_emb, ((0, 0), (0, PADW - EMB)))
    rest_p = jnp.pad(rest_emb, ((0, 0), (0, PADW - EMB)))
    u, s = _sc_gather_pool(ninp, user_p, rest_p)
    return _tc_mlp(u, s, W1, b1, W2, b2, W3, b3)


# speedup vs baseline: 1.1896x; 1.1386x over previous
import jax
import jax.numpy as jnp
import numpy as np
from jax import lax
from jax.experimental import pallas as pl
from jax.experimental.pallas import tpu as pltpu
from jax.experimental.pallas import tpu_sc as plsc

EMB = 64
HIST = 50
ROW = 1 + HIST
PAIR = 2 * ROW
PADW = 128
PADP = 112
NC, NS = 2, 16
NW = NC * NS
LANES = 16
NCH = EMB // LANES
UCH = 128

_DCOL = np.arange(PADP).reshape(PADP // LANES, LANES)
_HALF = (_DCOL >= ROW).astype(np.int32)
_CSRC = (_HALF * PADW + (_DCOL - _HALF * ROW)).astype(np.int32)


def _sc_mesh_kernel(body, out_type, scratch_types):
    return pl.kernel(
        body,
        out_type=out_type,
        mesh=plsc.VectorSubcoreMesh(core_axis_name="c", subcore_axis_name="s"),
        compiler_params=pltpu.CompilerParams(
            use_tc_tiling_on_sc=False, needs_layout_passes=False),
        scratch_types=scratch_types,
    )


def _sc_fmt(ninp):
    B = ninp.shape[0]
    BPW = B // NW
    NP = BPW // 2
    NQ = BPW // UCH
    PPQ = UCH // 2

    def body(nin_hbm, out_nin, out_uidx, slab_v, nin_v, uidx_v):
        wid = lax.axis_index("s") * NC + lax.axis_index("c")
        base = wid * BPW
        lane = lax.iota(jnp.int32, LANES)
        zero16 = jnp.zeros((LANES,), jnp.int32)
        chalf, ccol = [], []
        for c in range(PADP // LANES):
            dcol = lane + c * LANES
            half = jnp.where(dcol >= ROW, 1, 0).astype(jnp.int32)
            chalf.append(half)
            ccol.append(dcol - half * ROW)

        for q in range(NQ):
            pltpu.sync_copy(nin_hbm.at[pl.ds(base + q * UCH, UCH)], slab_v)

            @pl.loop(0, PPQ, unroll=2)
            def _(p):
                dbase = (q * PPQ + p) * PADP
                for c in range(PADP // LANES):
                    v = plsc.load_gather(slab_v, [2 * p + chalf[c], ccol[c]])
                    nin_v[pl.ds(dbase + c * LANES, LANES)] = v

            for m in range(UCH // LANES):
                u16 = plsc.load_gather(slab_v, [m * LANES + lane, zero16])
                uidx_v[q, pl.ds(m * LANES, LANES)] = u16

        pltpu.sync_copy(nin_v, out_nin.at[wid])
        pltpu.sync_copy(uidx_v, out_uidx.at[wid])

    f = _sc_mesh_kernel(
        body,
        (jax.ShapeDtypeStruct((NW, NP * PADP), jnp.int32),
         jax.ShapeDtypeStruct((NW, NQ, UCH), jnp.int32)),
        [
            pltpu.VMEM((UCH, PADW), jnp.int32),
            pltpu.VMEM((NP * PADP,), jnp.int32),
            pltpu.VMEM((NQ, UCH), jnp.int32),
        ],
    )
    return f(ninp)


def _sc_user(uidx_all, user_p, B):
    BPW = B // NW
    NQ = BPW // UCH

    def body(uidx_hbm, user_hbm, out_u, uidx_v, bufA, bufB, semA, semB):
        wid = lax.axis_index("s") * NC + lax.axis_index("c")
        base = wid * BPW
        pltpu.sync_copy(uidx_hbm.at[wid], uidx_v)
        bufs = (bufA, bufB)
        sems = (semA, semB)
        pltpu.async_copy(user_hbm.at[uidx_v.at[0]], bufA, semA)
        for q in range(NQ):
            b = q % 2
            if q + 1 < NQ:
                pltpu.async_copy(user_hbm.at[uidx_v.at[q + 1]],
                                 bufs[(q + 1) % 2], sems[(q + 1) % 2])
            pltpu.make_async_copy(user_hbm.at[uidx_v.at[q]],
                                  bufs[b], sems[b]).wait()
            pltpu.sync_copy(bufs[b].at[pl.ds(0, UCH), pl.ds(0, EMB)],
                            out_u.at[pl.ds(base + q * UCH, UCH)])

    f = _sc_mesh_kernel(
        body,
        jax.ShapeDtypeStruct((B, EMB), jnp.float32),
        [
            pltpu.VMEM((NQ, UCH), jnp.int32),
            pltpu.VMEM((UCH, PADW), jnp.float32),
            pltpu.VMEM((UCH, PADW), jnp.float32),
            pltpu.SemaphoreType.DMA,
            pltpu.SemaphoreType.DMA,
        ],
    )
    return f(uidx_all, user_p)


def _sc_rest(nin_c, rest_p, B):
    BPW = B // NW
    NP = BPW // 2

    def body(nin_hbm, rest_hbm, out_s, nin_v, acc, bufA, bufB, semA, semB):
        wid = lax.axis_index("s") * NC + lax.axis_index("c")
        base = wid * BPW
        pltpu.sync_copy(nin_hbm.at[wid], nin_v)
        bufs = (bufA, bufB)
        sems = (semA, semB)
        pltpu.async_copy(rest_hbm.at[nin_v.at[pl.ds(0, PAIR)]],
                         bufA.at[pl.ds(0, PAIR)], semA)

        @pl.loop(0, NP, step=2)
        def _(g):
            for b in range(2):
                t = g + b
                nb = (b + 1) % 2

                @pl.when(t + 1 < NP)
                def _():
                    pltpu.async_copy(
                        rest_hbm.at[nin_v.at[pl.ds((t + 1) * PADP, PAIR)]],
                        bufs[nb].at[pl.ds(0, PAIR)], sems[nb])

                pltpu.make_async_copy(
                    rest_hbm.at[nin_v.at[pl.ds(t * PADP, PAIR)]],
                    bufs[b].at[pl.ds(0, PAIR)], sems[b]).wait()
                buf = bufs[b]
                for half in range(2):
                    rb = half * ROW
                    arow = 2 * t + half
                    for c in range(NCH):
                        sl = pl.ds(c * LANES, LANES)
                        acc[arow, sl] = buf[rb + 1, sl]

                    @pl.loop(2, ROW, unroll=7)
                    def _(r):
                        for c in range(NCH):
                            sl = pl.ds(c * LANES, LANES)
                            plsc.addupdate(acc.at[arow, sl], buf[rb + r, sl])

        pltpu.async_copy(acc, out_s.at[pl.ds(base, BPW)], semA).wait()

    f = _sc_mesh_kernel(
        body,
        jax.ShapeDtypeStruct((B, EMB), jnp.float32),
        [
            pltpu.VMEM((B // NW // 2 * PADP,), jnp.int32),
            pltpu.VMEM((B // NW, EMB), jnp.float32),
            pltpu.VMEM((PAIR, PADW), jnp.float32),
            pltpu.VMEM((PAIR, PADW), jnp.float32),
            pltpu.SemaphoreType.DMA,
            pltpu.SemaphoreType.DMA,
        ],
    )
    return f(nin_c, rest_p)


def _mlp_body(u_ref, s_ref, w1u_ref, w1r_ref, b1_ref, w2_ref, b2_ref,
              w3_ref, b3_ref, o_ref):
    h1 = jnp.dot(u_ref[...], w1u_ref[...], preferred_element_type=jnp.float32)
    h1 += jnp.dot(s_ref[...], w1r_ref[...],
                  preferred_element_type=jnp.float32) * (1.0 / HIST)
    h1 = jnp.maximum(h1 + b1_ref[...], 0.0)
    h2 = jnp.dot(h1, w2_ref[...], preferred_element_type=jnp.float32)
    h2 = jnp.maximum(h2 + b2_ref[...], 0.0)
    y = jnp.dot(h2, w3_ref[...], preferred_element_type=jnp.float32)
    o_ref[...] = jax.nn.sigmoid(y + b3_ref[...])


def _tc_mlp(u, s, W1, b1, W2, b2, W3, b3):
    B = u.shape[0]
    H1, H2 = W1.shape[0], W2.shape[0]
    BLK = 2048
    grid = (B // BLK,)
    w1u = W1[:, :EMB].T
    w1r = W1[:, EMB:].T
    fixed = lambda i: (0, 0)
    return pl.pallas_call(
        _mlp_body,
        grid=grid,
        in_specs=[
            pl.BlockSpec((BLK, EMB), lambda i: (i, 0)),
            pl.BlockSpec((BLK, EMB), lambda i: (i, 0)),
            pl.BlockSpec((EMB, H1), fixed),
            pl.BlockSpec((EMB, H1), fixed),
            pl.BlockSpec((1, H1), fixed),
            pl.BlockSpec((H1, H2), fixed),
            pl.BlockSpec((1, H2), fixed),
            pl.BlockSpec((H2, 1), fixed),
            pl.BlockSpec((1, 1), fixed),
        ],
        out_specs=pl.BlockSpec((BLK, 1), lambda i: (i, 0)),
        out_shape=jax.ShapeDtypeStruct((B, 1), jnp.float32),
        compiler_params=pltpu.CompilerParams(
            dimension_semantics=("parallel",)),
    )(u, s, w1u, w1r, b1[None, :], W2.T, b2[None, :], W3.T, b3[None, :])


def kernel(net_input, user_emb, rest_emb, W1, b1, W2, b2, W3, b3):
    B = net_input.shape[0]
    B = net_input.shape[0]
    ninp = jnp.pad(net_input, ((0, 0), (0, PADW - ROW)))
    user_p = jnp.pad(user_emb, ((0, 0), (0, PADW - EMB)))
    rest_p = jnp.pad(rest_emb, ((0, 0), (0, PADW - EMB)))
    nin_c, uidx_all = _sc_fmt(ninp)
    s = _sc_rest(nin_c, rest_p, B)
    u = _sc_user(uidx_all, user_p, B)
    return _tc_mlp(u, s, W1, b1, W2, b2, W3, b3)


# speedup vs baseline: 1.1997x; 1.0085x over previous
import jax
import jax.numpy as jnp
from jax import lax
from jax.experimental import pallas as pl
from jax.experimental.pallas import tpu as pltpu
from jax.experimental.pallas import tpu_sc as plsc

EMB = 64
HIST = 50
ROW = 1 + HIST
PAIR = 2 * ROW
PADW = 128
PADP = 112
NC, NS = 2, 16
NW = NC * NS
LANES = 16
NCH = EMB // LANES
UCH = 128


def _sc_mesh_kernel(body, out_type, scratch_types):
    return pl.kernel(
        body,
        out_type=out_type,
        mesh=plsc.VectorSubcoreMesh(core_axis_name="c", subcore_axis_name="s"),
        compiler_params=pltpu.CompilerParams(
            use_tc_tiling_on_sc=False, needs_layout_passes=False),
        scratch_types=scratch_types,
    )


def _sc_fmt(ninp):
    B = ninp.shape[0]
    BPW = B // NW
    NP = BPW // 2
    NQ = BPW // UCH
    PPQ = UCH // 2

    def body(nin_hbm, out_nin, out_uidx, slab_v, nin_v, uidx_v):
        wid = lax.axis_index("s") * NC + lax.axis_index("c")
        base = wid * BPW
        lane = lax.iota(jnp.int32, LANES)
        zero16 = jnp.zeros((LANES,), jnp.int32)
        chalf, ccol = [], []
        for c in range(PADP // LANES):
            dcol = lane + c * LANES
            half = jnp.where(dcol >= ROW, 1, 0).astype(jnp.int32)
            chalf.append(half)
            ccol.append(dcol - half * ROW)

        for q in range(NQ):
            pltpu.sync_copy(nin_hbm.at[pl.ds(base + q * UCH, UCH)], slab_v)

            @pl.loop(0, PPQ, unroll=2)
            def _(p):
                dbase = (q * PPQ + p) * PADP
                for c in range(PADP // LANES):
                    v = plsc.load_gather(slab_v, [2 * p + chalf[c], ccol[c]])
                    nin_v[pl.ds(dbase + c * LANES, LANES)] = v

            for m in range(UCH // LANES):
                u16 = plsc.load_gather(slab_v, [m * LANES + lane, zero16])
                uidx_v[q, pl.ds(m * LANES, LANES)] = u16

        pltpu.sync_copy(nin_v, out_nin.at[wid])
        pltpu.sync_copy(uidx_v, out_uidx.at[wid])

    f = _sc_mesh_kernel(
        body,
        (jax.ShapeDtypeStruct((NW, NP * PADP), jnp.int32),
         jax.ShapeDtypeStruct((NW, NQ, UCH), jnp.int32)),
        [
            pltpu.VMEM((UCH, PADW), jnp.int32),
            pltpu.VMEM((NP * PADP,), jnp.int32),
            pltpu.VMEM((NQ, UCH), jnp.int32),
        ],
    )
    return f(ninp)


def _sc_user(uidx_all, user_p, B):
    BPW = B // NW
    NQ = BPW // UCH

    def body(uidx_hbm, user_hbm, out_u, uidx_v, bufA, bufB, semA, semB):
        wid = lax.axis_index("s") * NC + lax.axis_index("c")
        base = wid * BPW
        pltpu.sync_copy(uidx_hbm.at[wid], uidx_v)
        bufs = (bufA, bufB)
        sems = (semA, semB)
        pltpu.async_copy(user_hbm.at[uidx_v.at[0]], bufA, semA)
        for q in range(NQ):
            b = q % 2
            if q + 1 < NQ:
                pltpu.async_copy(user_hbm.at[uidx_v.at[q + 1]],
                                 bufs[(q + 1) % 2], sems[(q + 1) % 2])
            pltpu.make_async_copy(user_hbm.at[uidx_v.at[q]],
                                  bufs[b], sems[b]).wait()
            pltpu.sync_copy(bufs[b].at[pl.ds(0, UCH), pl.ds(0, EMB)],
                            out_u.at[pl.ds(base + q * UCH, UCH)])

    f = _sc_mesh_kernel(
        body,
        jax.ShapeDtypeStruct((B, EMB), jnp.float32),
        [
            pltpu.VMEM((NQ, UCH), jnp.int32),
            pltpu.VMEM((UCH, PADW), jnp.float32),
            pltpu.VMEM((UCH, PADW), jnp.float32),
            pltpu.SemaphoreType.DMA,
            pltpu.SemaphoreType.DMA,
        ],
    )
    return f(uidx_all, user_p)


def _sc_rest(nin_c, rest_p, B):
    BPW = B // NW
    NP = BPW // 2

    def body(nin_hbm, rest_hbm, out_s, nin_v, acc, bufA, bufB, semA, semB):
        wid = lax.axis_index("s") * NC + lax.axis_index("c")
        base = wid * BPW
        pltpu.sync_copy(nin_hbm.at[wid], nin_v)
        bufs = (bufA, bufB)
        sems = (semA, semB)
        pltpu.async_copy(rest_hbm.at[nin_v.at[pl.ds(0, PAIR)]],
                         bufA.at[pl.ds(0, PAIR)], semA)

        @pl.loop(0, NP, step=2)
        def _(g):
            for b in range(2):
                t = g + b
                nb = (b + 1) % 2

                @pl.when(t + 1 < NP)
                def _():
                    pltpu.async_copy(
                        rest_hbm.at[nin_v.at[pl.ds((t + 1) * PADP, PAIR)]],
                        bufs[nb].at[pl.ds(0, PAIR)], sems[nb])

                pltpu.make_async_copy(
                    rest_hbm.at[nin_v.at[pl.ds(t * PADP, PAIR)]],
                    bufs[b].at[pl.ds(0, PAIR)], sems[b]).wait()
                buf = bufs[b]
                for half in range(2):
                    rb = half * ROW
                    arow = 2 * t + half
                    for c in range(NCH):
                        sl = pl.ds(c * LANES, LANES)
                        acc[arow, sl] = buf[rb + 1, sl]

                    @pl.loop(2, ROW, unroll=7)
                    def _(r):
                        for c in range(NCH):
                            sl = pl.ds(c * LANES, LANES)
                            plsc.addupdate(acc.at[arow, sl], buf[rb + r, sl])

        pltpu.async_copy(acc, out_s.at[pl.ds(base, BPW)], semA).wait()

    f = _sc_mesh_kernel(
        body,
        jax.ShapeDtypeStruct((B, EMB), jnp.float32),
        [
            pltpu.VMEM((B // NW // 2 * PADP,), jnp.int32),
            pltpu.VMEM((B // NW, EMB), jnp.float32),
            pltpu.VMEM((PAIR, PADW), jnp.float32),
            pltpu.VMEM((PAIR, PADW), jnp.float32),
            pltpu.SemaphoreType.DMA,
            pltpu.SemaphoreType.DMA,
        ],
    )
    return f(nin_c, rest_p)


def _mlp_body(u_ref, s_ref, w1u_ref, w1r_ref, b1_ref, w2_ref, b2_ref,
              w3_ref, b3_ref, o_ref):
    h1 = jnp.dot(u_ref[...], w1u_ref[...], preferred_element_type=jnp.float32)
    h1 += jnp.dot(s_ref[...], w1r_ref[...],
                  preferred_element_type=jnp.float32) * (1.0 / HIST)
    h1 = jnp.maximum(h1 + b1_ref[...], 0.0)
    h2 = jnp.dot(h1, w2_ref[...], preferred_element_type=jnp.float32)
    h2 = jnp.maximum(h2 + b2_ref[...], 0.0)
    y = jnp.dot(h2, w3_ref[...], preferred_element_type=jnp.float32)
    o_ref[...] = jax.nn.sigmoid(y + b3_ref[...])


def _tc_mlp(u, s, W1, b1, W2, b2, W3, b3):
    B = u.shape[0]
    H1, H2 = W1.shape[0], W2.shape[0]
    BLK = 2048
    grid = (B // BLK,)
    w1u = W1[:, :EMB].T
    w1r = W1[:, EMB:].T
    fixed = lambda i: (0, 0)
    return pl.pallas_call(
        _mlp_body,
        grid=grid,
        in_specs=[
            pl.BlockSpec((BLK, EMB), lambda i: (i, 0)),
            pl.BlockSpec((BLK, EMB), lambda i: (i, 0)),
            pl.BlockSpec((EMB, H1), fixed),
            pl.BlockSpec((EMB, H1), fixed),
            pl.BlockSpec((1, H1), fixed),
            pl.BlockSpec((H1, H2), fixed),
            pl.BlockSpec((1, H2), fixed),
            pl.BlockSpec((H2, 1), fixed),
            pl.BlockSpec((1, 1), fixed),
        ],
        out_specs=pl.BlockSpec((BLK, 1), lambda i: (i, 0)),
        out_shape=jax.ShapeDtypeStruct((B, 1), jnp.float32),
        compiler_params=pltpu.CompilerParams(
            dimension_semantics=("parallel",)),
    )(u, s, w1u, w1r, b1[None, :], W2.T, b2[None, :], W3.T, b3[None, :])


def kernel(net_input, user_emb, rest_emb, W1, b1, W2, b2, W3, b3):
    B = net_input.shape[0]
    B = net_input.shape[0]
    ninp = jnp.pad(net_input, ((0, 0), (0, PADW - ROW)))
    user_p = jnp.pad(user_emb, ((0, 0), (0, PADW - EMB)))
    rest_p = jnp.pad(rest_emb, ((0, 0), (0, PADW - EMB)))
    nin_c, uidx_all = _sc_fmt(ninp)
    s = _sc_rest(nin_c, rest_p, B)
    u = _sc_user(uidx_all, user_p, B)
    return _tc_mlp(u, s, W1, b1, W2, b2, W3, b3)
